# Initial kernel scaffold; baseline (speedup 1.0000x reference)
import jax
import jax.numpy as jnp
from jax.experimental import pallas as pl


def kernel(x, emb, W, b, gamma, beta):
    raise NotImplementedError("write your pallas kernel here")


# speedup vs baseline: 2.7635x; 2.7635x over previous
import functools

import jax
import jax.numpy as jnp
from jax import lax
from jax.experimental import pallas as pl
from jax.experimental.pallas import tpu as pltpu
from jax.experimental.pallas import tpu_sc as plsc


def _table_body(emb_ref, w_ref, b_ref, g_ref, beta_ref, out_ref):
    h = lax.dot_general(
        emb_ref[...], w_ref[...], (((1,), (1,)), ((), ())),
        preferred_element_type=jnp.float32,
    )
    h = h + b_ref[...]
    mu = jnp.mean(h, axis=-1, keepdims=True)
    d = h - mu
    var = jnp.mean(d * d, axis=-1, keepdims=True)
    t = (d * lax.rsqrt(var + 1e-5)) * g_ref[...] + beta_ref[...]
    V, D = t.shape
    left = jnp.broadcast_to(t[:, None, :], (V, V, D))
    right = jnp.broadcast_to(t[None, :, :], (V, V, D))
    out_ref[...] = jnp.concatenate([left, right], axis=-1)


def _make_paired_table(emb, W, b, gamma, beta):
    V, D = emb.shape
    paired = pl.pallas_call(
        _table_body,
        out_shape=jax.ShapeDtypeStruct((V, V, 2 * D), jnp.float32),
    )(emb, W, b.reshape(1, D), gamma.reshape(1, D), beta.reshape(1, D))
    return paired.reshape(V * V, 2 * D)


def _make_gather(N2, n_workers, chunk):
    n_per_w = N2 // n_workers
    n_chunks = n_per_w // chunk
    mesh = plsc.VectorSubcoreMesh(core_axis_name="c", subcore_axis_name="s")

    @functools.partial(
        pl.kernel,
        out_type=jax.ShapeDtypeStruct((N2, 128), jnp.float32),
        mesh=mesh,
        scratch_types=[
            pltpu.VMEM((chunk,), jnp.int32),
            pltpu.VMEM((chunk, 128), jnp.float32),
            pltpu.SemaphoreType.DMA,
        ],
    )
    def gather_k(tab_hbm, idx_hbm, out_hbm, idx_v, rows_v, sem):
        wid = lax.axis_index("s") * 2 + lax.axis_index("c")
        base = wid * n_per_w

        def body(i, carry):
            off = pl.multiple_of(base + i * chunk, chunk)
            pltpu.sync_copy(idx_hbm.at[pl.ds(off, chunk)], idx_v)
            pltpu.async_copy(tab_hbm.at[idx_v], rows_v, sem).wait()
            pltpu.sync_copy(rows_v, out_hbm.at[pl.ds(off, chunk)])
            return carry

        lax.fori_loop(0, n_chunks, body, 0)

    return gather_k


def kernel(x, emb, W, b, gamma, beta):
    B, L = x.shape
    V, D = emb.shape
    N2 = (B * L) // 2
    paired = _make_paired_table(emb, W, b, gamma, beta)
    xf = x.astype(jnp.int32).reshape(N2, 2)
    idx2 = xf[:, 0] * V + xf[:, 1]
    gather = _make_gather(N2, n_workers=32, chunk=128)
    out2 = gather(paired, idx2)
    return out2.reshape(B, L, D)


# speedup vs baseline: 3.9016x; 1.4118x over previous
import functools

import jax
import jax.numpy as jnp
from jax import lax
from jax.experimental import pallas as pl
from jax.experimental.pallas import tpu as pltpu
from jax.experimental.pallas import tpu_sc as plsc


def _table_body(emb_ref, w_ref, b_ref, g_ref, beta_ref, out_ref):
    h = lax.dot_general(
        emb_ref[...], w_ref[...], (((1,), (1,)), ((), ())),
        preferred_element_type=jnp.float32,
    )
    h = h + b_ref[...]
    mu = jnp.mean(h, axis=-1, keepdims=True)
    d = h - mu
    var = jnp.mean(d * d, axis=-1, keepdims=True)
    t = (d * lax.rsqrt(var + 1e-5)) * g_ref[...] + beta_ref[...]
    V, D = t.shape
    left = jnp.broadcast_to(t[:, None, :], (V, V, D))
    right = jnp.broadcast_to(t[None, :, :], (V, V, D))
    out_ref[...] = jnp.concatenate([left, right], axis=-1)


def _make_paired_table(emb, W, b, gamma, beta):
    V, D = emb.shape
    paired = pl.pallas_call(
        _table_body,
        out_shape=jax.ShapeDtypeStruct((V, V, 2 * D), jnp.float32),
    )(emb, W, b.reshape(1, D), gamma.reshape(1, D), beta.reshape(1, D))
    return paired.reshape(V * V, 2 * D)


_NSLOT = 4


def _make_gather(N2, VV, n_workers, chunk):
    n_per_w = N2 // n_workers
    n_chunks = n_per_w // chunk
    n_outer = n_chunks // _NSLOT
    mesh = plsc.VectorSubcoreMesh(core_axis_name="c", subcore_axis_name="s")

    scratch = (
        [pltpu.VMEM_SHARED((VV, 128), jnp.float32)]
        + [pltpu.VMEM((chunk,), jnp.int32) for _ in range(_NSLOT)]
        + [pltpu.VMEM((chunk, 128), jnp.float32) for _ in range(_NSLOT)]
        + [pltpu.SemaphoreType.DMA for _ in range(2 * _NSLOT + 1)]
    )

    @functools.partial(
        pl.kernel,
        out_type=jax.ShapeDtypeStruct((N2, 128), jnp.float32),
        mesh=mesh,
        scratch_types=scratch,
    )
    def gather_k(tab_hbm, idx_hbm, out_hbm, *refs):
        tab_v = refs[0]
        idx_vs = refs[1:1 + _NSLOT]
        row_vs = refs[1 + _NSLOT:1 + 2 * _NSLOT]
        sem_i = refs[1 + 2 * _NSLOT:1 + 3 * _NSLOT]
        sem_s = refs[1 + 3 * _NSLOT:1 + 4 * _NSLOT]
        sem_g = refs[1 + 4 * _NSLOT]

        wid = lax.axis_index("s") * 2 + lax.axis_index("c")
        base = wid * n_per_w

        @pl.when(lax.axis_index("s") == 0)
        def _():
            pltpu.sync_copy(tab_hbm, tab_v)

        plsc.subcore_barrier()
        for b in range(_NSLOT):
            pltpu.async_copy(
                idx_hbm.at[pl.ds(base + b * chunk, chunk)], idx_vs[b], sem_i[b]
            )

        def outer(i, carry):
            g0 = i * _NSLOT
            for b in range(_NSLOT):
                g = g0 + b
                off = base + g * chunk
                pltpu.make_async_copy(
                    idx_hbm.at[pl.ds(off, chunk)], idx_vs[b], sem_i[b]
                ).wait()
                @pl.when(i > 0)
                def _():
                    pltpu.make_async_copy(
                        row_vs[b], out_hbm.at[pl.ds(off, chunk)], sem_s[b]
                    ).wait()
                pltpu.async_copy(tab_v.at[idx_vs[b]], row_vs[b], sem_g).wait()
                @pl.when(g + _NSLOT < n_chunks)
                def _():
                    pltpu.async_copy(
                        idx_hbm.at[pl.ds(off + _NSLOT * chunk, chunk)],
                        idx_vs[b],
                        sem_i[b],
                    )
                pltpu.async_copy(
                    row_vs[b], out_hbm.at[pl.ds(off, chunk)], sem_s[b]
                )
            return carry

        lax.fori_loop(0, n_outer, outer, 0)

        for b in range(_NSLOT):
            off = base + (n_chunks - _NSLOT + b) * chunk
            pltpu.make_async_copy(
                row_vs[b], out_hbm.at[pl.ds(off, chunk)], sem_s[b]
            ).wait()

    return gather_k


def kernel(x, emb, W, b, gamma, beta):
    B, L = x.shape
    V, D = emb.shape
    N2 = (B * L) // 2
    paired = _make_paired_table(emb, W, b, gamma, beta)
    xf = x.astype(jnp.int32).reshape(N2, 2)
    idx2 = xf[:, 0] * V + xf[:, 1]
    gather = _make_gather(N2, V * V, n_workers=32, chunk=128)
    out2 = gather(paired, idx2)
    return out2.reshape(B, L, D)
